# baseline (speedup 1.0000x reference)
import functools

import jax
import jax.numpy as jnp
from jax import lax
from jax.experimental import pallas as pl
from jax.experimental.pallas import tpu as pltpu
from jax.experimental.pallas import tpu_sc as plsc

_H = 16
_N2 = 256 * 256
_V = 961
_VPAD = 976
_VH = _VPAD * _H

_NC = 2
_NS = 16
_L = 16
_NW = _NC * _NS
_CHUNK = (_H * _N2) // _NW


def _sc_gather_body(table_hbm, idx_hbm, out_hbm, table_v, table_h, idx_v, out_v):
    cid = lax.axis_index("c")
    sid = lax.axis_index("s")
    wid = sid * _NC + cid
    h = wid // 2
    col0 = (wid % 2) * _CHUNK
    pltpu.sync_copy(table_hbm, table_v)
    pltpu.sync_copy(idx_hbm.at[pl.ds(col0, _CHUNK)], idx_v)

    lane16 = lax.iota(jnp.int32, _L) * _H
    for t in range(_VPAD // _L):
        th = plsc.load_gather(table_v, [lane16 + (t * _L * _H + h)])
        table_h[pl.ds(t * _L, _L)] = th

    @plsc.parallel_loop(0, _CHUNK // _L, unroll=16)
    def body(g):
        iv = idx_v[pl.ds(g * _L, _L)]
        vals = plsc.load_gather(table_h, [iv])
        out_v[pl.ds(g * _L, _L)] = vals
    pltpu.sync_copy(out_v, out_hbm.at[pl.ds(h * _N2 + col0, _CHUNK)])


_sc_gather = functools.partial(
    pl.kernel,
    out_type=jax.ShapeDtypeStruct((_H * _N2,), jnp.float32),
    mesh=plsc.VectorSubcoreMesh(core_axis_name="c", subcore_axis_name="s"),
    compiler_params=pltpu.CompilerParams(needs_layout_passes=False),
    scratch_types=[
        pltpu.VMEM((_VH,), jnp.float32),
        pltpu.VMEM((_VPAD,), jnp.float32),
        pltpu.VMEM((_CHUNK,), jnp.int32),
        pltpu.VMEM((_CHUNK,), jnp.float32),
    ],
)(_sc_gather_body)


def _add_body(b_hbm, x_ref, o_ref, b_vmem, sem):
    @pl.when((pl.program_id(0) == 0) & (pl.program_id(1) == 0))
    def _():
        pltpu.make_async_copy(b_hbm, b_vmem, sem).start()
        pltpu.make_async_copy(b_hbm, b_vmem, sem).wait()

    hb = x_ref.shape[1]
    j = pl.program_id(1)
    o_ref[...] = x_ref[...] + b_vmem[pl.ds(j * hb, hb)]


def _tc_add(x, bias_flat):
    b, h, n, _ = x.shape
    br = bias_flat.reshape(h, n, n)
    hb = 8
    return pl.pallas_call(
        _add_body,
        grid=(b, h // hb),
        in_specs=[
            pl.BlockSpec(memory_space=pl.ANY),
            pl.BlockSpec((1, hb, n, n), lambda i, j: (i, j, 0, 0)),
        ],
        out_specs=pl.BlockSpec((1, hb, n, n), lambda i, j: (i, j, 0, 0)),
        out_shape=jax.ShapeDtypeStruct(x.shape, jnp.float32),
        scratch_shapes=[
            pltpu.VMEM((h, n, n), jnp.float32),
            pltpu.SemaphoreType.DMA,
        ],
    )(br, x)


def kernel(x, relative_position_bias_table, relative_position_index):
    table_flat = jnp.pad(
        relative_position_bias_table, ((0, _VPAD - _V), (0, 0))
    ).reshape(-1)
    idx_flat = relative_position_index.reshape(-1).astype(jnp.int32)
    bias_flat = _sc_gather(table_flat, idx_flat)
    return _tc_add(x, bias_flat)


# speedup vs baseline: 1.0891x; 1.0891x over previous
import functools

import jax
import jax.numpy as jnp
from jax import lax
from jax.experimental import pallas as pl
from jax.experimental.pallas import tpu as pltpu
from jax.experimental.pallas import tpu_sc as plsc

_H = 16
_N2 = 256 * 256
_V = 961
_VPAD = 976
_VH = _VPAD * _H

_NC = 2
_NS = 16
_L = 16
_NW = _NC * _NS
_CHUNK = (_H * _N2) // _NW


def _sc_gather_body(table_hbm, idx_hbm, out_hbm, table_v, table_h, idx_v, out_v):
    cid = lax.axis_index("c")
    sid = lax.axis_index("s")
    wid = sid * _NC + cid
    h = wid // 2
    col0 = (wid % 2) * _CHUNK
    pltpu.sync_copy(table_hbm, table_v)
    pltpu.sync_copy(idx_hbm.at[pl.ds(col0, _CHUNK)], idx_v)

    lane16 = lax.iota(jnp.int32, _L) * _H
    for t in range(_VPAD // _L):
        th = plsc.load_gather(table_v, [lane16 + (t * _L * _H + h)])
        table_h[pl.ds(t * _L, _L)] = th

    @plsc.parallel_loop(0, _CHUNK // _L, unroll=16)
    def body(g):
        iv = idx_v[pl.ds(g * _L, _L)]
        vals = plsc.load_gather(table_h, [iv])
        out_v[pl.ds(g * _L, _L)] = vals
    pltpu.sync_copy(out_v, out_hbm.at[pl.ds(h * _N2 + col0, _CHUNK)])


_sc_gather = functools.partial(
    pl.kernel,
    out_type=jax.ShapeDtypeStruct((_H * _N2,), jnp.float32),
    mesh=plsc.VectorSubcoreMesh(core_axis_name="c", subcore_axis_name="s"),
    compiler_params=pltpu.CompilerParams(needs_layout_passes=False),
    scratch_types=[
        pltpu.VMEM((_VH,), jnp.float32),
        pltpu.VMEM((_VPAD,), jnp.float32),
        pltpu.VMEM((_CHUNK,), jnp.int32),
        pltpu.VMEM((_CHUNK,), jnp.float32),
    ],
)(_sc_gather_body)


def _add_body(b_hbm, x_ref, o_ref, b_vmem, sem):
    @pl.when(pl.program_id(0) == 0)
    def _():
        pltpu.make_async_copy(b_hbm, b_vmem, sem).start()
        pltpu.make_async_copy(b_hbm, b_vmem, sem).wait()

    o_ref[...] = x_ref[...] + b_vmem[...]


def _tc_add(x, bias_flat):
    b, h, n, _ = x.shape
    br = bias_flat.reshape(h, n, n)
    bb = 2
    return pl.pallas_call(
        _add_body,
        grid=(b // bb,),
        in_specs=[
            pl.BlockSpec(memory_space=pl.ANY),
            pl.BlockSpec((bb, h, n, n), lambda i: (i, 0, 0, 0)),
        ],
        out_specs=pl.BlockSpec((bb, h, n, n), lambda i: (i, 0, 0, 0)),
        out_shape=jax.ShapeDtypeStruct(x.shape, jnp.float32),
        scratch_shapes=[
            pltpu.VMEM((h, n, n), jnp.float32),
            pltpu.SemaphoreType.DMA,
        ],
    )(br, x)


def kernel(x, relative_position_bias_table, relative_position_index):
    table_flat = jnp.pad(
        relative_position_bias_table, ((0, _VPAD - _V), (0, 0))
    ).reshape(-1)
    idx_flat = relative_position_index.reshape(-1).astype(jnp.int32)
    bias_flat = _sc_gather(table_flat, idx_flat)
    return _tc_add(x, bias_flat)


# speedup vs baseline: 1.1169x; 1.0255x over previous
import functools

import jax
import jax.numpy as jnp
from jax import lax
from jax.experimental import pallas as pl
from jax.experimental.pallas import tpu as pltpu
from jax.experimental.pallas import tpu_sc as plsc

_H = 16
_N2 = 256 * 256
_V = 961
_VPAD = 976
_VH = _VPAD * _H

_NC = 2
_NS = 16
_L = 16
_NW = _NC * _NS
_CHUNK = (_H * _N2) // _NW


_KCHUNK = _N2 // _NW


def _sc_gather_body(table_hbm, idx_hbm, out_hbm, table_v, idx_v, out_v):
    cid = lax.axis_index("c")
    sid = lax.axis_index("s")
    wid = sid * _NC + cid
    kb = wid * _KCHUNK
    pltpu.sync_copy(table_hbm, table_v)
    pltpu.sync_copy(idx_hbm.at[pl.ds(kb, _KCHUNK)], idx_v)

    @plsc.parallel_loop(0, _KCHUNK // _L, unroll=2)
    def body(g):
        iv = idx_v[pl.ds(g * _L, _L)]
        for h in range(_H):
            vals = plsc.load_gather(table_v, [iv + (h * _VPAD)])
            out_v[h, pl.ds(g * _L, _L)] = vals

    pltpu.sync_copy(out_v, out_hbm.at[:, pl.ds(kb, _KCHUNK)])


_sc_gather = functools.partial(
    pl.kernel,
    out_type=jax.ShapeDtypeStruct((_H, _N2), jnp.float32),
    mesh=plsc.VectorSubcoreMesh(core_axis_name="c", subcore_axis_name="s"),
    compiler_params=pltpu.CompilerParams(needs_layout_passes=False),
    scratch_types=[
        pltpu.VMEM((_VH,), jnp.float32),
        pltpu.VMEM((_KCHUNK,), jnp.int32),
        pltpu.VMEM((_H, _KCHUNK), jnp.float32),
    ],
)(_sc_gather_body)


def _add_body(b_hbm, x_ref, o_ref, b_vmem, sem):
    @pl.when(pl.program_id(0) == 0)
    def _():
        pltpu.make_async_copy(b_hbm, b_vmem, sem).start()
        pltpu.make_async_copy(b_hbm, b_vmem, sem).wait()

    o_ref[...] = x_ref[...] + b_vmem[...]


def _tc_add(x, bias_flat):
    b, h, n, _ = x.shape
    br = bias_flat.reshape(h, n, n)
    bb = 2
    return pl.pallas_call(
        _add_body,
        grid=(b // bb,),
        in_specs=[
            pl.BlockSpec(memory_space=pl.ANY),
            pl.BlockSpec((bb, h, n, n), lambda i: (i, 0, 0, 0)),
        ],
        out_specs=pl.BlockSpec((bb, h, n, n), lambda i: (i, 0, 0, 0)),
        out_shape=jax.ShapeDtypeStruct(x.shape, jnp.float32),
        scratch_shapes=[
            pltpu.VMEM((h, n, n), jnp.float32),
            pltpu.SemaphoreType.DMA,
        ],
    )(br, x)


def kernel(x, relative_position_bias_table, relative_position_index):
    table_t = jnp.pad(
        relative_position_bias_table, ((0, _VPAD - _V), (0, 0))
    ).T.reshape(-1)
    idx_flat = relative_position_index.reshape(-1).astype(jnp.int32)
    bias_flat = _sc_gather(table_t, idx_flat)
    return _tc_add(x, bias_flat)
